# initial kernel scaffold (unmeasured)
import jax
import jax.numpy as jnp
from jax import lax
from jax.experimental import pallas as pl
from jax.experimental.pallas import tpu as pltpu

N_DEV = 8
B, SQ, D_MODEL = 2, 512, 768
HQ_LOCAL, DH = 8, 64
DQ = HQ_LOCAL * DH
ROWS = B * SQ
CHUNK = ROWS // N_DEV


def kernel(x, Wq, K_ext, V_ext, Wo):
    def body(x_ref, wq_ref, k_ref, v_ref, wo_ref, out_ref,
             acc_ref, comm_ref, send_sems, recv_sems):
        my = lax.axis_index("i")
        right = lax.rem(my + 1, N_DEV)

        xv = x_ref[...].astype(jnp.bfloat16).reshape(ROWS, D_MODEL)
        wq = wq_ref[:, pl.ds(my * DQ, DQ)].astype(jnp.bfloat16)
        q = lax.dot_general(
            xv, wq, (((1,), (0,)), ((), ())),
            preferred_element_type=jnp.float32,
        ).astype(jnp.bfloat16)

        qi = lax.broadcasted_iota(jnp.int32, (SQ, SQ), 0)
        ki = lax.broadcasted_iota(jnp.int32, (SQ, SQ), 1)
        mask = (jnp.abs(qi - ki) <= 128) | (ki < 32) | (qi < 32)

        ctx_rows = []
        for b in range(B):
            qb = q[b * SQ:(b + 1) * SQ, :]
            heads = []
            for h in range(HQ_LOCAL):
                qbh = qb[:, h * DH:(h + 1) * DH]
                kbh = k_ref[b, :, h, :].astype(jnp.bfloat16)
                s = lax.dot_general(
                    qbh, kbh, (((1,), (1,)), ((), ())),
                    preferred_element_type=jnp.float32,
                ) * 0.125
                s = jnp.where(mask, s, -1e9)
                s = s - jnp.max(s, axis=1, keepdims=True)
                w = jnp.exp(s)
                w = w / jnp.sum(w, axis=1, keepdims=True)
                vbh = v_ref[b, :, h, :].astype(jnp.bfloat16)
                heads.append(lax.dot_general(
                    w.astype(jnp.bfloat16), vbh, (((1,), (0,)), ((), ())),
                    preferred_element_type=jnp.float32,
                ))
            ctx_rows.append(jnp.concatenate(heads, axis=1))
        ctx = jnp.concatenate(ctx_rows, axis=0).astype(jnp.bfloat16)

        wo = wo_ref[pl.ds(my * DQ, DQ), :].astype(jnp.bfloat16)
        acc_ref[...] = lax.dot_general(
            ctx, wo, (((1,), (0,)), ((), ())),
            preferred_element_type=jnp.float32,
        )

        for s_ in range(N_DEV - 1):
            send_c = lax.rem(my + N_DEV - s_, N_DEV)
            recv_c = lax.rem(my + N_DEV - 1 - s_, N_DEV)
            rdma = pltpu.make_async_remote_copy(
                src_ref=acc_ref.at[pl.ds(send_c * CHUNK, CHUNK)],
                dst_ref=comm_ref.at[s_],
                send_sem=send_sems.at[s_],
                recv_sem=recv_sems.at[s_],
                device_id=(right,),
                device_id_type=pl.DeviceIdType.MESH,
            )
            rdma.start()
            rdma.wait()
            idx = pl.ds(recv_c * CHUNK, CHUNK)
            acc_ref[idx, :] = acc_ref[idx, :] + comm_ref[s_]

        for t in range(N_DEV - 1):
            send_c = lax.rem(my + 1 + N_DEV - t, N_DEV)
            rdma = pltpu.make_async_remote_copy(
                src_ref=acc_ref.at[pl.ds(send_c * CHUNK, CHUNK)],
                dst_ref=acc_ref.at[pl.ds(send_c * CHUNK, CHUNK)],
                send_sem=send_sems.at[N_DEV - 1 + t],
                recv_sem=recv_sems.at[N_DEV - 1 + t],
                device_id=(right,),
                device_id_type=pl.DeviceIdType.MESH,
            )
            rdma.start()
            rdma.wait()

        out_ref[...] = acc_ref[...].reshape(B, SQ, D_MODEL)

    return pl.pallas_call(
        body,
        out_shape=jax.ShapeDtypeStruct((B, SQ, D_MODEL), jnp.float32),
        in_specs=[pl.BlockSpec(memory_space=pltpu.VMEM)] * 5,
        out_specs=pl.BlockSpec(memory_space=pltpu.VMEM),
        scratch_shapes=[
            pltpu.VMEM((ROWS, D_MODEL), jnp.float32),
            pltpu.VMEM((N_DEV - 1, CHUNK, D_MODEL), jnp.float32),
            pltpu.SemaphoreType.DMA((2 * (N_DEV - 1),)),
            pltpu.SemaphoreType.DMA((2 * (N_DEV - 1),)),
        ],
        compiler_params=pltpu.CompilerParams(collective_id=0),
    )(x, Wq, K_ext, V_ext, Wo)


# baseline (device time: 129851 ns/iter reference)
import jax
import jax.numpy as jnp
from jax import lax
from jax.experimental import pallas as pl
from jax.experimental.pallas import tpu as pltpu

N_DEV = 8
B, SQ, D_MODEL = 2, 512, 768
HQ_LOCAL, DH = 8, 64
DQ = HQ_LOCAL * DH
ROWS = B * SQ
CHUNK = ROWS // N_DEV


def kernel(x, Wq, K_ext, V_ext, Wo):
    def body(x_ref, wq_ref, k_ref, v_ref, wo_ref, out_ref,
             acc_ref, comm_ref, send_sems, recv_sems):
        my = lax.axis_index("i")
        right = lax.rem(my + 1, N_DEV)
        left = lax.rem(my + N_DEV - 1, N_DEV)

        barrier_sem = pltpu.get_barrier_semaphore()
        for nbr in (left, right):
            pl.semaphore_signal(
                barrier_sem, inc=1,
                device_id=(nbr,), device_id_type=pl.DeviceIdType.MESH,
            )
        pl.semaphore_wait(barrier_sem, 2)

        xv = x_ref[...].astype(jnp.bfloat16).reshape(ROWS, D_MODEL)
        wq = wq_ref[:, pl.ds(my * DQ, DQ)].astype(jnp.bfloat16)
        q = lax.dot_general(
            xv, wq, (((1,), (0,)), ((), ())),
            preferred_element_type=jnp.float32,
        ).astype(jnp.bfloat16)

        qi = lax.broadcasted_iota(jnp.int32, (SQ, SQ), 0)
        ki = lax.broadcasted_iota(jnp.int32, (SQ, SQ), 1)
        mask = (jnp.abs(qi - ki) <= 128) | (ki < 32) | (qi < 32)

        ctx_rows = []
        for b in range(B):
            qb = q[b * SQ:(b + 1) * SQ, :]
            heads = []
            for h in range(HQ_LOCAL):
                qbh = qb[:, h * DH:(h + 1) * DH]
                kbh = k_ref[b, :, h, :].astype(jnp.bfloat16)
                s = lax.dot_general(
                    qbh, kbh, (((1,), (1,)), ((), ())),
                    preferred_element_type=jnp.float32,
                ) * 0.125
                s = jnp.where(mask, s, -1e9)
                s = s - jnp.max(s, axis=1, keepdims=True)
                w = jnp.exp(s)
                w = w / jnp.sum(w, axis=1, keepdims=True)
                vbh = v_ref[b, :, h, :].astype(jnp.bfloat16)
                heads.append(lax.dot_general(
                    w.astype(jnp.bfloat16), vbh, (((1,), (0,)), ((), ())),
                    preferred_element_type=jnp.float32,
                ))
            ctx_rows.append(jnp.concatenate(heads, axis=1))
        ctx = jnp.concatenate(ctx_rows, axis=0).astype(jnp.bfloat16)

        wo = wo_ref[pl.ds(my * DQ, DQ), :].astype(jnp.bfloat16)
        acc_ref[...] = lax.dot_general(
            ctx, wo, (((1,), (0,)), ((), ())),
            preferred_element_type=jnp.float32,
        )

        for s_ in range(N_DEV - 1):
            send_c = lax.rem(my + N_DEV - s_, N_DEV)
            recv_c = lax.rem(my + N_DEV - 1 - s_, N_DEV)
            rdma = pltpu.make_async_remote_copy(
                src_ref=acc_ref.at[pl.ds(send_c * CHUNK, CHUNK)],
                dst_ref=comm_ref.at[s_],
                send_sem=send_sems.at[s_],
                recv_sem=recv_sems.at[s_],
                device_id=(right,),
                device_id_type=pl.DeviceIdType.MESH,
            )
            rdma.start()
            rdma.wait()
            idx = pl.ds(recv_c * CHUNK, CHUNK)
            acc_ref[idx, :] = acc_ref[idx, :] + comm_ref[s_]

        for t in range(N_DEV - 1):
            send_c = lax.rem(my + 1 + N_DEV - t, N_DEV)
            rdma = pltpu.make_async_remote_copy(
                src_ref=acc_ref.at[pl.ds(send_c * CHUNK, CHUNK)],
                dst_ref=acc_ref.at[pl.ds(send_c * CHUNK, CHUNK)],
                send_sem=send_sems.at[N_DEV - 1 + t],
                recv_sem=recv_sems.at[N_DEV - 1 + t],
                device_id=(right,),
                device_id_type=pl.DeviceIdType.MESH,
            )
            rdma.start()
            rdma.wait()

        out_ref[...] = acc_ref[...].reshape(B, SQ, D_MODEL)

    return pl.pallas_call(
        body,
        out_shape=jax.ShapeDtypeStruct((B, SQ, D_MODEL), jnp.float32),
        in_specs=[pl.BlockSpec(memory_space=pltpu.VMEM)] * 5,
        out_specs=pl.BlockSpec(memory_space=pltpu.VMEM),
        scratch_shapes=[
            pltpu.VMEM((ROWS, D_MODEL), jnp.float32),
            pltpu.VMEM((N_DEV - 1, CHUNK, D_MODEL), jnp.float32),
            pltpu.SemaphoreType.DMA((2 * (N_DEV - 1),)),
            pltpu.SemaphoreType.DMA((2 * (N_DEV - 1),)),
        ],
        compiler_params=pltpu.CompilerParams(
            collective_id=0, vmem_limit_bytes=96 * 1024 * 1024,
        ),
    )(x, Wq, K_ext, V_ext, Wo)


# device time: 76031 ns/iter; 1.7079x vs baseline; 1.7079x over previous
import jax
import jax.numpy as jnp
from jax import lax
from jax.experimental import pallas as pl
from jax.experimental.pallas import tpu as pltpu

N_DEV = 8
B, SQ, D_MODEL = 2, 512, 768
HQ_LOCAL, DH = 8, 64
DQ = HQ_LOCAL * DH
ROWS = B * SQ
CHUNK = ROWS // N_DEV


def kernel(x, Wq, K_ext, V_ext, Wo):
    def body(x_ref, wq_ref, k_ref, v_ref, wo_ref, out_ref,
             acc_ref, sb_ref, rb1_ref, rb2_ref, rb3_ref, res_ref,
             send_sems, recv_sems):
        r = lax.axis_index("i")
        yb = lax.rem(lax.div(r, 2), 2)
        xb = lax.rem(lax.rem(r, 2) + yb, 2)
        zb = lax.div(r, 4)
        px = r + 1 - 2 * lax.rem(r, 2)
        py = 4 * zb + 3 - lax.rem(r, 4)
        pz = lax.rem(r + 4, N_DEV)

        barrier_sem = pltpu.get_barrier_semaphore()
        for nbr in (px, py, pz):
            pl.semaphore_signal(
                barrier_sem, inc=1,
                device_id=(nbr,), device_id_type=pl.DeviceIdType.MESH,
            )
        pl.semaphore_wait(barrier_sem, 3)

        wq = wq_ref[:, pl.ds(r * DQ, DQ)].astype(jnp.bfloat16)
        wo = wo_ref[pl.ds(r * DQ, DQ), :].astype(jnp.bfloat16)
        qi = lax.broadcasted_iota(jnp.int32, (SQ, SQ), 0)
        ki = lax.broadcasted_iota(jnp.int32, (SQ, SQ), 1)
        mask = (jnp.abs(qi - ki) <= 128) | (ki < 32) | (qi < 32)

        def compute_batch(b):
            xv = x_ref[pl.ds(b, 1), :, :].reshape(SQ, D_MODEL)
            q = lax.dot_general(
                xv.astype(jnp.bfloat16), wq, (((1,), (0,)), ((), ())),
                preferred_element_type=jnp.float32,
            ).astype(jnp.bfloat16)
            heads = []
            for h in range(HQ_LOCAL):
                qbh = q[:, h * DH:(h + 1) * DH]
                kbh = k_ref[pl.ds(b, 1), :, h, :].reshape(SQ, DH)
                s = lax.dot_general(
                    qbh, kbh.astype(jnp.bfloat16), (((1,), (1,)), ((), ())),
                    preferred_element_type=jnp.float32,
                ) * 0.125
                s = jnp.where(mask, s, -1e9)
                s = s - jnp.max(s, axis=1, keepdims=True)
                w = jnp.exp(s)
                w = w / jnp.sum(w, axis=1, keepdims=True)
                vbh = v_ref[pl.ds(b, 1), :, h, :].reshape(SQ, DH)
                heads.append(lax.dot_general(
                    w.astype(jnp.bfloat16), vbh.astype(jnp.bfloat16),
                    (((1,), (0,)), ((), ())),
                    preferred_element_type=jnp.float32,
                ))
            ctx = jnp.concatenate(heads, axis=1).astype(jnp.bfloat16)
            acc_ref[pl.ds(b * SQ, SQ), :] = lax.dot_general(
                ctx, wo, (((1,), (0,)), ((), ())),
                preferred_element_type=jnp.float32,
            )

        g1 = (1 - xb) * (4 * CHUNK)
        k1 = xb * (4 * CHUNK)
        compute_batch(1 - xb)
        sb_ref[pl.ds(g1, 4 * CHUNK), :] = (
            acc_ref[pl.ds(g1, 4 * CHUNK), :].astype(jnp.bfloat16))
        rs1 = pltpu.make_async_remote_copy(
            src_ref=sb_ref.at[pl.ds(g1, 4 * CHUNK)],
            dst_ref=rb1_ref,
            send_sem=send_sems.at[0], recv_sem=recv_sems.at[0],
            device_id=(px,), device_id_type=pl.DeviceIdType.MESH,
        )
        rs1.start()
        compute_batch(xb)
        rs1.wait()
        acc_ref[pl.ds(k1, 4 * CHUNK), :] = (
            acc_ref[pl.ds(k1, 4 * CHUNK), :] + rb1_ref[...].astype(jnp.float32))

        g2 = (4 * xb + 2 * (1 - yb)) * CHUNK
        k2 = (4 * xb + 2 * yb) * CHUNK
        sb_ref[pl.ds(g2, 2 * CHUNK), :] = (
            acc_ref[pl.ds(g2, 2 * CHUNK), :].astype(jnp.bfloat16))
        rs2 = pltpu.make_async_remote_copy(
            src_ref=sb_ref.at[pl.ds(g2, 2 * CHUNK)],
            dst_ref=rb2_ref,
            send_sem=send_sems.at[1], recv_sem=recv_sems.at[1],
            device_id=(py,), device_id_type=pl.DeviceIdType.MESH,
        )
        rs2.start()
        rs2.wait()
        acc_ref[pl.ds(k2, 2 * CHUNK), :] = (
            acc_ref[pl.ds(k2, 2 * CHUNK), :] + rb2_ref[...].astype(jnp.float32))

        g3 = (4 * xb + 2 * yb + (1 - zb)) * CHUNK
        k3 = (4 * xb + 2 * yb + zb) * CHUNK
        sb_ref[pl.ds(g3, CHUNK), :] = (
            acc_ref[pl.ds(g3, CHUNK), :].astype(jnp.bfloat16))
        rs3 = pltpu.make_async_remote_copy(
            src_ref=sb_ref.at[pl.ds(g3, CHUNK)],
            dst_ref=rb3_ref,
            send_sem=send_sems.at[2], recv_sem=recv_sems.at[2],
            device_id=(pz,), device_id_type=pl.DeviceIdType.MESH,
        )
        rs3.start()
        rs3.wait()
        res_ref[pl.ds(k3, CHUNK), :] = (
            acc_ref[pl.ds(k3, CHUNK), :]
            + rb3_ref[...].astype(jnp.float32)).astype(jnp.bfloat16)

        ag1 = pltpu.make_async_remote_copy(
            src_ref=res_ref.at[pl.ds(k3, CHUNK)],
            dst_ref=res_ref.at[pl.ds(k3, CHUNK)],
            send_sem=send_sems.at[3], recv_sem=recv_sems.at[3],
            device_id=(pz,), device_id_type=pl.DeviceIdType.MESH,
        )
        ag1.start()
        ag1.wait()
        ag2 = pltpu.make_async_remote_copy(
            src_ref=res_ref.at[pl.ds(k2, 2 * CHUNK)],
            dst_ref=res_ref.at[pl.ds(k2, 2 * CHUNK)],
            send_sem=send_sems.at[4], recv_sem=recv_sems.at[4],
            device_id=(py,), device_id_type=pl.DeviceIdType.MESH,
        )
        ag2.start()
        ag2.wait()
        ag3 = pltpu.make_async_remote_copy(
            src_ref=res_ref.at[pl.ds(k1, 4 * CHUNK)],
            dst_ref=res_ref.at[pl.ds(k1, 4 * CHUNK)],
            send_sem=send_sems.at[5], recv_sem=recv_sems.at[5],
            device_id=(px,), device_id_type=pl.DeviceIdType.MESH,
        )
        ag3.start()
        ag3.wait()

        out_ref[...] = res_ref[...].astype(jnp.float32).reshape(B, SQ, D_MODEL)

    return pl.pallas_call(
        body,
        out_shape=jax.ShapeDtypeStruct((B, SQ, D_MODEL), jnp.float32),
        in_specs=[pl.BlockSpec(memory_space=pltpu.VMEM)] * 5,
        out_specs=pl.BlockSpec(memory_space=pltpu.VMEM),
        scratch_shapes=[
            pltpu.VMEM((ROWS, D_MODEL), jnp.float32),
            pltpu.VMEM((ROWS, D_MODEL), jnp.bfloat16),
            pltpu.VMEM((4 * CHUNK, D_MODEL), jnp.bfloat16),
            pltpu.VMEM((2 * CHUNK, D_MODEL), jnp.bfloat16),
            pltpu.VMEM((CHUNK, D_MODEL), jnp.bfloat16),
            pltpu.VMEM((ROWS, D_MODEL), jnp.bfloat16),
            pltpu.SemaphoreType.DMA((6,)),
            pltpu.SemaphoreType.DMA((6,)),
        ],
        compiler_params=pltpu.CompilerParams(
            collective_id=0, vmem_limit_bytes=96 * 1024 * 1024,
        ),
    )(x, Wq, K_ext, V_ext, Wo)


# device time: 68240 ns/iter; 1.9029x vs baseline; 1.1142x over previous
import jax
import jax.numpy as jnp
from jax import lax
from jax.experimental import pallas as pl
from jax.experimental.pallas import tpu as pltpu

N_DEV = 8
B, SQ, D_MODEL = 2, 512, 768
HQ_LOCAL, DH = 8, 64
DQ = HQ_LOCAL * DH
ROWS = B * SQ
CHUNK = ROWS // N_DEV
NC = 3
CW = D_MODEL // NC


def kernel(x, Wq, K_ext, V_ext, Wo):
    def body(x_ref, wq_ref, k_ref, v_ref, wo_ref, out_ref,
             acc_ref, sb_ref, rb1_ref, rb2_ref, rb3_ref, res_ref,
             send_sems, recv_sems):
        r = lax.axis_index("i")
        yb = lax.rem(lax.div(r, 2), 2)
        xb = lax.rem(lax.rem(r, 2) + yb, 2)
        zb = lax.div(r, 4)
        px = r + 1 - 2 * lax.rem(r, 2)
        py = 4 * zb + 3 - lax.rem(r, 4)
        pz = lax.rem(r + 4, N_DEV)

        barrier_sem = pltpu.get_barrier_semaphore()
        for nbr in (px, py, pz):
            pl.semaphore_signal(
                barrier_sem, inc=1,
                device_id=(nbr,), device_id_type=pl.DeviceIdType.MESH,
            )
        pl.semaphore_wait(barrier_sem, 3)

        wq = wq_ref[:, pl.ds(r * DQ, DQ)].astype(jnp.bfloat16)
        wo = wo_ref[pl.ds(r * DQ, DQ), :].astype(jnp.bfloat16)
        qi = lax.broadcasted_iota(jnp.int32, (SQ, SQ), 0)
        ki = lax.broadcasted_iota(jnp.int32, (SQ, SQ), 1)
        mask = (jnp.abs(qi - ki) <= 128) | (ki < 32) | (qi < 32)
        bias = jnp.where(mask, 0.0, -1e9).astype(jnp.float32)

        def compute_batch(b):
            xv = x_ref[pl.ds(b, 1), :, :].reshape(SQ, D_MODEL)
            q = lax.dot_general(
                xv.astype(jnp.bfloat16), wq, (((1,), (0,)), ((), ())),
                preferred_element_type=jnp.float32,
            )
            q = (q * 0.125).astype(jnp.bfloat16)
            heads = []
            for h in range(HQ_LOCAL):
                qbh = q[:, h * DH:(h + 1) * DH]
                kbh = k_ref[pl.ds(b, 1), :, h, :].reshape(SQ, DH)
                s = lax.dot_general(
                    qbh, kbh.astype(jnp.bfloat16), (((1,), (1,)), ((), ())),
                    preferred_element_type=jnp.float32,
                ) + bias
                w = jnp.exp(s)
                w = w / jnp.sum(w, axis=1, keepdims=True)
                vbh = v_ref[pl.ds(b, 1), :, h, :].reshape(SQ, DH)
                heads.append(lax.dot_general(
                    w.astype(jnp.bfloat16), vbh.astype(jnp.bfloat16),
                    (((1,), (0,)), ((), ())),
                    preferred_element_type=jnp.float32,
                ))
            ctx = jnp.concatenate(heads, axis=1).astype(jnp.bfloat16)
            acc_ref[pl.ds(b * SQ, SQ), :] = lax.dot_general(
                ctx, wo, (((1,), (0,)), ((), ())),
                preferred_element_type=jnp.float32,
            )

        axes = [(px, xb), (py, yb), (pz, zb)]
        rbufs = [rb1_ref, rb2_ref, rb3_ref]

        class Color:
            def __init__(self, g):
                self.g = g
                self.col = g * CW
                a = [axes[(g + k) % NC] for k in range(NC)]
                self.p = [a[0][0], a[1][0], a[2][0]]
                b1, b2, b3 = a[0][1], a[1][1], a[2][1]
                self.k = [None, None, None]
                self.gv = [None, None, None]
                self.k[0] = b1 * (4 * CHUNK)
                self.gv[0] = (1 - b1) * (4 * CHUNK)
                self.k[1] = self.k[0] + b2 * (2 * CHUNK)
                self.gv[1] = self.k[0] + (1 - b2) * (2 * CHUNK)
                self.k[2] = self.k[1] + b3 * CHUNK
                self.gv[2] = self.k[1] + (1 - b3) * CHUNK
                self.n = [4 * CHUNK, 2 * CHUNK, CHUNK]
                self.rs = [None, None, None]
                self.ag = [None, None, None]

            def rs_start(self, s):
                cs = pl.ds(self.col, CW)
                rows = pl.ds(self.gv[s], self.n[s])
                sb_ref[rows, cs] = acc_ref[rows, cs].astype(jnp.bfloat16)
                self.rs[s] = pltpu.make_async_remote_copy(
                    src_ref=sb_ref.at[rows, cs],
                    dst_ref=rbufs[s].at[pl.ds(0, self.n[s]), cs],
                    send_sem=send_sems.at[self.g * 6 + s],
                    recv_sem=recv_sems.at[self.g * 6 + s],
                    device_id=(self.p[s],),
                    device_id_type=pl.DeviceIdType.MESH,
                )
                self.rs[s].start()

            def rs_finish(self, s):
                cs = pl.ds(self.col, CW)
                self.rs[s].wait()
                rows = pl.ds(self.k[s], self.n[s])
                got = rbufs[s][pl.ds(0, self.n[s]), cs].astype(jnp.float32)
                if s < 2:
                    acc_ref[rows, cs] = acc_ref[rows, cs] + got
                else:
                    res_ref[rows, cs] = (
                        acc_ref[rows, cs] + got).astype(jnp.bfloat16)

            def ag_start(self, s):
                rows = pl.ds(self.k[2 - s], self.n[2 - s])
                cs = pl.ds(self.col, CW)
                self.ag[s] = pltpu.make_async_remote_copy(
                    src_ref=res_ref.at[rows, cs],
                    dst_ref=res_ref.at[rows, cs],
                    send_sem=send_sems.at[self.g * 6 + 3 + s],
                    recv_sem=recv_sems.at[self.g * 6 + 3 + s],
                    device_id=(self.p[2 - s],),
                    device_id_type=pl.DeviceIdType.MESH,
                )
                self.ag[s].start()

            def ag_finish(self, s):
                self.ag[s].wait()

        c0, c1, c2 = Color(0), Color(1), Color(2)

        compute_batch(1 - xb)
        c0.rs_start(0)
        compute_batch(xb)

        c0.rs_finish(0)
        c1.rs_start(0)
        c2.rs_start(0)
        c0.rs_start(1)
        c1.rs_finish(0)
        c0.rs_finish(1)
        c0.rs_start(2)
        c1.rs_start(1)
        c2.rs_finish(0)
        c2.rs_start(1)
        c0.rs_finish(2)
        c0.ag_start(0)
        c1.rs_finish(1)
        c1.rs_start(2)
        c2.rs_finish(1)
        c2.rs_start(2)
        c0.ag_finish(0)
        c0.ag_start(1)
        c1.rs_finish(2)
        c1.ag_start(0)
        c2.rs_finish(2)
        c2.ag_start(0)
        c0.ag_finish(1)
        c0.ag_start(2)
        c1.ag_finish(0)
        c1.ag_start(1)
        c2.ag_finish(0)
        c2.ag_start(1)
        c0.ag_finish(2)
        c1.ag_finish(1)
        c1.ag_start(2)
        c2.ag_finish(1)
        c2.ag_start(2)
        c1.ag_finish(2)
        c2.ag_finish(2)

        out_ref[...] = res_ref[...].astype(jnp.float32).reshape(B, SQ, D_MODEL)

    return pl.pallas_call(
        body,
        out_shape=jax.ShapeDtypeStruct((B, SQ, D_MODEL), jnp.float32),
        in_specs=[pl.BlockSpec(memory_space=pltpu.VMEM)] * 5,
        out_specs=pl.BlockSpec(memory_space=pltpu.VMEM),
        scratch_shapes=[
            pltpu.VMEM((ROWS, D_MODEL), jnp.float32),
            pltpu.VMEM((ROWS, D_MODEL), jnp.bfloat16),
            pltpu.VMEM((4 * CHUNK, D_MODEL), jnp.bfloat16),
            pltpu.VMEM((2 * CHUNK, D_MODEL), jnp.bfloat16),
            pltpu.VMEM((CHUNK, D_MODEL), jnp.bfloat16),
            pltpu.VMEM((ROWS, D_MODEL), jnp.bfloat16),
            pltpu.SemaphoreType.DMA((18,)),
            pltpu.SemaphoreType.DMA((18,)),
        ],
        compiler_params=pltpu.CompilerParams(
            collective_id=0, vmem_limit_bytes=96 * 1024 * 1024,
        ),
    )(x, Wq, K_ext, V_ext, Wo)


# device time: 57298 ns/iter; 2.2662x vs baseline; 1.1910x over previous
import jax
import jax.numpy as jnp
from jax import lax
from jax.experimental import pallas as pl
from jax.experimental.pallas import tpu as pltpu

N_DEV = 8
B, SQ, D_MODEL = 2, 512, 768
HQ_LOCAL, DH = 8, 64
DQ = HQ_LOCAL * DH
ROWS = B * SQ
CHUNK = ROWS // N_DEV
NC = 3
CW = D_MODEL // NC


def kernel(x, Wq, K_ext, V_ext, Wo):
    def body(x_ref, wq_ref, k_ref, v_ref, wo_ref, out_ref,
             acc_ref, rb1_ref, rb2_ref, rb3_ref, send_sems, recv_sems):
        r = lax.axis_index("i")
        yb = lax.rem(lax.div(r, 2), 2)
        xb = lax.rem(lax.rem(r, 2) + yb, 2)
        zb = lax.div(r, 4)
        px = r + 1 - 2 * lax.rem(r, 2)
        py = 4 * zb + 3 - lax.rem(r, 4)
        pz = lax.rem(r + 4, N_DEV)

        barrier_sem = pltpu.get_barrier_semaphore()
        for nbr in (px, py, pz):
            pl.semaphore_signal(
                barrier_sem, inc=1,
                device_id=(nbr,), device_id_type=pl.DeviceIdType.MESH,
            )
        pl.semaphore_wait(barrier_sem, 3)

        wq = wq_ref[:, pl.ds(r * DQ, DQ)].astype(jnp.bfloat16)
        wo = wo_ref[pl.ds(r * DQ, DQ), :].astype(jnp.bfloat16)
        qi = lax.broadcasted_iota(jnp.int32, (SQ, SQ), 0)
        ki = lax.broadcasted_iota(jnp.int32, (SQ, SQ), 1)
        mask = (jnp.abs(qi - ki) <= 128) | (ki < 32) | (qi < 32)
        bias = jnp.where(mask, 0.0, -1e9).astype(jnp.float32)

        def compute_batch(b):
            xv = x_ref[pl.ds(b, 1), :, :].reshape(SQ, D_MODEL)
            q = lax.dot_general(
                xv.astype(jnp.bfloat16), wq, (((1,), (0,)), ((), ())),
                preferred_element_type=jnp.float32,
            )
            q = (q * 0.125).astype(jnp.bfloat16)
            heads = []
            for h in range(HQ_LOCAL):
                qbh = q[:, h * DH:(h + 1) * DH]
                kbh = k_ref[pl.ds(b, 1), :, h * DH:(h + 1) * DH].reshape(SQ, DH)
                s = lax.dot_general(
                    qbh, kbh.astype(jnp.bfloat16), (((1,), (1,)), ((), ())),
                    preferred_element_type=jnp.float32,
                ) + bias
                w = jnp.exp(s)
                recip = 1.0 / jnp.sum(w, axis=1, keepdims=True)
                vbh = v_ref[pl.ds(b, 1), :, h * DH:(h + 1) * DH].reshape(SQ, DH)
                ctx_h = lax.dot_general(
                    w.astype(jnp.bfloat16), vbh.astype(jnp.bfloat16),
                    (((1,), (0,)), ((), ())),
                    preferred_element_type=jnp.float32,
                ) * recip
                heads.append(ctx_h.astype(jnp.bfloat16))
            ctx = jnp.concatenate(heads, axis=1)
            acc_ref[pl.ds(b * SQ, SQ), :] = lax.dot_general(
                ctx, wo, (((1,), (0,)), ((), ())),
                preferred_element_type=jnp.float32,
            ).astype(jnp.bfloat16)

        axes = [(px, xb), (py, yb), (pz, zb)]
        rbufs = [rb1_ref, rb2_ref, rb3_ref]

        class Color:
            def __init__(self, g):
                self.g = g
                self.col = g * CW
                a = [axes[(g + k) % NC] for k in range(NC)]
                self.p = [a[0][0], a[1][0], a[2][0]]
                b1, b2, b3 = a[0][1], a[1][1], a[2][1]
                self.k = [None, None, None]
                self.gv = [None, None, None]
                self.k[0] = b1 * (4 * CHUNK)
                self.gv[0] = (1 - b1) * (4 * CHUNK)
                self.k[1] = self.k[0] + b2 * (2 * CHUNK)
                self.gv[1] = self.k[0] + (1 - b2) * (2 * CHUNK)
                self.k[2] = self.k[1] + b3 * CHUNK
                self.gv[2] = self.k[1] + (1 - b3) * CHUNK
                self.n = [4 * CHUNK, 2 * CHUNK, CHUNK]
                self.rs = [None, None, None]
                self.ag = [None, None, None]

            def rs_start(self, s):
                cs = pl.ds(self.col, CW)
                self.rs[s] = pltpu.make_async_remote_copy(
                    src_ref=acc_ref.at[pl.ds(self.gv[s], self.n[s]), cs],
                    dst_ref=rbufs[s].at[pl.ds(0, self.n[s]), cs],
                    send_sem=send_sems.at[self.g * 6 + s],
                    recv_sem=recv_sems.at[self.g * 6 + s],
                    device_id=(self.p[s],),
                    device_id_type=pl.DeviceIdType.MESH,
                )
                self.rs[s].start()

            def rs_finish(self, s):
                cs = pl.ds(self.col, CW)
                self.rs[s].wait()
                rows = pl.ds(self.k[s], self.n[s])
                acc_ref[rows, cs] = (
                    acc_ref[rows, cs] + rbufs[s][pl.ds(0, self.n[s]), cs])

            def ag_start(self, s):
                rows = pl.ds(self.k[2 - s], self.n[2 - s])
                cs = pl.ds(self.col, CW)
                self.ag[s] = pltpu.make_async_remote_copy(
                    src_ref=acc_ref.at[rows, cs],
                    dst_ref=acc_ref.at[rows, cs],
                    send_sem=send_sems.at[self.g * 6 + 3 + s],
                    recv_sem=recv_sems.at[self.g * 6 + 3 + s],
                    device_id=(self.p[2 - s],),
                    device_id_type=pl.DeviceIdType.MESH,
                )
                self.ag[s].start()

            def ag_finish(self, s):
                self.ag[s].wait()

            def store(self):
                cs = pl.ds(self.col, CW)
                out_ref[:, :, cs] = (
                    acc_ref[:, cs].astype(jnp.float32).reshape(B, SQ, CW))

        c0, c1, c2 = Color(0), Color(1), Color(2)

        compute_batch(1 - xb)
        c0.rs_start(0)
        compute_batch(xb)

        c0.rs_finish(0)
        c1.rs_start(0)
        c2.rs_start(0)
        c0.rs_start(1)
        c1.rs_finish(0)
        c0.rs_finish(1)
        c0.rs_start(2)
        c1.rs_start(1)
        c2.rs_finish(0)
        c2.rs_start(1)
        c0.rs_finish(2)
        c0.ag_start(0)
        c1.rs_finish(1)
        c1.rs_start(2)
        c2.rs_finish(1)
        c2.rs_start(2)
        c0.ag_finish(0)
        c0.ag_start(1)
        c1.rs_finish(2)
        c1.ag_start(0)
        c2.rs_finish(2)
        c2.ag_start(0)
        c0.ag_finish(1)
        c0.ag_start(2)
        c1.ag_finish(0)
        c1.ag_start(1)
        c2.ag_finish(0)
        c2.ag_start(1)
        c0.ag_finish(2)
        c0.store()
        c1.ag_finish(1)
        c1.ag_start(2)
        c2.ag_finish(1)
        c2.ag_start(2)
        c1.ag_finish(2)
        c1.store()
        c2.ag_finish(2)
        c2.store()

    kv_shape = (B, SQ, HQ_LOCAL * DH)
    return pl.pallas_call(
        body,
        out_shape=jax.ShapeDtypeStruct((B, SQ, D_MODEL), jnp.float32),
        in_specs=[pl.BlockSpec(memory_space=pltpu.VMEM)] * 5,
        out_specs=pl.BlockSpec(memory_space=pltpu.VMEM),
        scratch_shapes=[
            pltpu.VMEM((ROWS, D_MODEL), jnp.bfloat16),
            pltpu.VMEM((4 * CHUNK, D_MODEL), jnp.bfloat16),
            pltpu.VMEM((2 * CHUNK, D_MODEL), jnp.bfloat16),
            pltpu.VMEM((CHUNK, D_MODEL), jnp.bfloat16),
            pltpu.SemaphoreType.DMA((18,)),
            pltpu.SemaphoreType.DMA((18,)),
        ],
        compiler_params=pltpu.CompilerParams(
            collective_id=0, vmem_limit_bytes=96 * 1024 * 1024,
        ),
    )(x, Wq, K_ext.reshape(kv_shape), V_ext.reshape(kv_shape), Wo)


# device time: 57085 ns/iter; 2.2747x vs baseline; 1.0037x over previous
import jax
import jax.numpy as jnp
from jax import lax
from jax.experimental import pallas as pl
from jax.experimental.pallas import tpu as pltpu

N_DEV = 8
B, SQ, D_MODEL = 2, 512, 768
HQ_LOCAL, DH = 8, 64
DQ = HQ_LOCAL * DH
ROWS = B * SQ
CHUNK = ROWS // N_DEV
NC = 3
CW = D_MODEL // NC


def kernel(x, Wq, K_ext, V_ext, Wo):
    def body(x_ref, wq_ref, k_ref, v_ref, wo_ref, out_ref,
             acc_ref, rb1_ref, rb2_ref, rb3_ref, send_sems, recv_sems):
        r = lax.axis_index("i")
        yb = lax.rem(lax.div(r, 2), 2)
        xb = lax.rem(lax.rem(r, 2) + yb, 2)
        zb = lax.div(r, 4)
        px = r + 1 - 2 * lax.rem(r, 2)
        py = 4 * zb + 3 - lax.rem(r, 4)
        pz = lax.rem(r + 4, N_DEV)

        barrier_sem = pltpu.get_barrier_semaphore()
        for nbr in (px, py, pz):
            pl.semaphore_signal(
                barrier_sem, inc=1,
                device_id=(nbr,), device_id_type=pl.DeviceIdType.MESH,
            )
        pl.semaphore_wait(barrier_sem, 3)

        wq = wq_ref[:, pl.ds(r * DQ, DQ)].astype(jnp.bfloat16)
        wo = wo_ref[pl.ds(r * DQ, DQ), :].astype(jnp.bfloat16)
        qi = lax.broadcasted_iota(jnp.int32, (SQ, SQ), 0)
        ki = lax.broadcasted_iota(jnp.int32, (SQ, SQ), 1)
        mask = (jnp.abs(qi - ki) <= 128) | (ki < 32) | (qi < 32)
        bias = jnp.where(mask, 0.0, -1e9).astype(jnp.float32)

        def compute_batch(b):
            xv = x_ref[pl.ds(b, 1), :, :].reshape(SQ, D_MODEL)
            q = lax.dot_general(
                xv.astype(jnp.bfloat16), wq, (((1,), (0,)), ((), ())),
                preferred_element_type=jnp.float32,
            )
            q = (q * 0.125).astype(jnp.bfloat16)
            heads = []
            for h in range(HQ_LOCAL):
                qbh = q[:, h * DH:(h + 1) * DH]
                kbh = k_ref[pl.ds(b, 1), :, h * DH:(h + 1) * DH].reshape(SQ, DH)
                s = lax.dot_general(
                    qbh, kbh.astype(jnp.bfloat16), (((1,), (1,)), ((), ())),
                    preferred_element_type=jnp.float32,
                ) + bias
                w = jnp.exp(s)
                recip = 1.0 / jnp.sum(w, axis=1, keepdims=True)
                vbh = v_ref[pl.ds(b, 1), :, h * DH:(h + 1) * DH].reshape(SQ, DH)
                ctx_h = lax.dot_general(
                    w.astype(jnp.bfloat16), vbh.astype(jnp.bfloat16),
                    (((1,), (0,)), ((), ())),
                    preferred_element_type=jnp.float32,
                ) * recip
                heads.append(ctx_h.astype(jnp.bfloat16))
            ctx = jnp.concatenate(heads, axis=1)
            part = lax.dot_general(
                ctx, wo, (((1,), (0,)), ((), ())),
                preferred_element_type=jnp.float32,
            )
            rows = pl.ds(b * SQ, SQ)
            for g in range(NC):
                acc_ref[g, rows, :] = (
                    part[:, g * CW:(g + 1) * CW].astype(jnp.bfloat16))

        axes = [(px, xb), (py, yb), (pz, zb)]
        rbufs = [rb1_ref, rb2_ref, rb3_ref]

        class Color:
            def __init__(self, g):
                self.g = g
                a = [axes[(g + k) % NC] for k in range(NC)]
                self.p = [a[0][0], a[1][0], a[2][0]]
                b1, b2, b3 = a[0][1], a[1][1], a[2][1]
                self.k = [b1 * (4 * CHUNK), None, None]
                self.gv = [(1 - b1) * (4 * CHUNK), None, None]
                self.k[1] = self.k[0] + b2 * (2 * CHUNK)
                self.gv[1] = self.k[0] + (1 - b2) * (2 * CHUNK)
                self.k[2] = self.k[1] + b3 * CHUNK
                self.gv[2] = self.k[1] + (1 - b3) * CHUNK
                self.n = [4 * CHUNK, 2 * CHUNK, CHUNK]
                self.rs = [None, None, None]
                self.ag = [None, None, None]

            def rs_start(self, s):
                self.rs[s] = pltpu.make_async_remote_copy(
                    src_ref=acc_ref.at[self.g, pl.ds(self.gv[s], self.n[s]), :],
                    dst_ref=rbufs[s].at[self.g, pl.ds(0, self.n[s]), :],
                    send_sem=send_sems.at[self.g * 6 + s],
                    recv_sem=recv_sems.at[self.g * 6 + s],
                    device_id=(self.p[s],),
                    device_id_type=pl.DeviceIdType.MESH,
                )
                self.rs[s].start()

            def rs_finish(self, s):
                self.rs[s].wait()
                rows = pl.ds(self.k[s], self.n[s])
                acc_ref[self.g, rows, :] = (
                    acc_ref[self.g, rows, :]
                    + rbufs[s][self.g, pl.ds(0, self.n[s]), :])

            def ag_start(self, s):
                rows = pl.ds(self.k[2 - s], self.n[2 - s])
                self.ag[s] = pltpu.make_async_remote_copy(
                    src_ref=acc_ref.at[self.g, rows, :],
                    dst_ref=acc_ref.at[self.g, rows, :],
                    send_sem=send_sems.at[self.g * 6 + 3 + s],
                    recv_sem=recv_sems.at[self.g * 6 + 3 + s],
                    device_id=(self.p[2 - s],),
                    device_id_type=pl.DeviceIdType.MESH,
                )
                self.ag[s].start()

            def ag_finish(self, s):
                self.ag[s].wait()

            def store(self):
                out_ref[:, :, pl.ds(self.g * CW, CW)] = (
                    acc_ref[self.g, :, :].astype(jnp.float32)
                    .reshape(B, SQ, CW))

        c0, c1, c2 = Color(0), Color(1), Color(2)

        compute_batch(1 - xb)
        c0.rs_start(0)
        compute_batch(xb)

        c0.rs_finish(0)
        c1.rs_start(0)
        c2.rs_start(0)
        c0.rs_start(1)
        c1.rs_finish(0)
        c0.rs_finish(1)
        c0.rs_start(2)
        c1.rs_start(1)
        c2.rs_finish(0)
        c2.rs_start(1)
        c0.rs_finish(2)
        c0.ag_start(0)
        c1.rs_finish(1)
        c1.rs_start(2)
        c2.rs_finish(1)
        c2.rs_start(2)
        c0.ag_finish(0)
        c0.ag_start(1)
        c1.rs_finish(2)
        c1.ag_start(0)
        c2.rs_finish(2)
        c2.ag_start(0)
        c0.ag_finish(1)
        c0.ag_start(2)
        c1.ag_finish(0)
        c1.ag_start(1)
        c2.ag_finish(0)
        c2.ag_start(1)
        c0.ag_finish(2)
        c0.store()
        c1.ag_finish(1)
        c1.ag_start(2)
        c2.ag_finish(1)
        c2.ag_start(2)
        c1.ag_finish(2)
        c1.store()
        c2.ag_finish(2)
        c2.store()

    kv_shape = (B, SQ, HQ_LOCAL * DH)
    return pl.pallas_call(
        body,
        out_shape=jax.ShapeDtypeStruct((B, SQ, D_MODEL), jnp.float32),
        in_specs=[pl.BlockSpec(memory_space=pltpu.VMEM)] * 5,
        out_specs=pl.BlockSpec(memory_space=pltpu.VMEM),
        scratch_shapes=[
            pltpu.VMEM((NC, ROWS, CW), jnp.bfloat16),
            pltpu.VMEM((NC, 4 * CHUNK, CW), jnp.bfloat16),
            pltpu.VMEM((NC, 2 * CHUNK, CW), jnp.bfloat16),
            pltpu.VMEM((NC, CHUNK, CW), jnp.bfloat16),
            pltpu.SemaphoreType.DMA((18,)),
            pltpu.SemaphoreType.DMA((18,)),
        ],
        compiler_params=pltpu.CompilerParams(
            collective_id=0, vmem_limit_bytes=96 * 1024 * 1024,
        ),
    )(x, Wq, K_ext.reshape(kv_shape), V_ext.reshape(kv_shape), Wo)


# device time: 52846 ns/iter; 2.4572x vs baseline; 1.0802x over previous
import jax
import jax.numpy as jnp
from jax import lax
from jax.experimental import pallas as pl
from jax.experimental.pallas import tpu as pltpu

N_DEV = 8
B, SQ, D_MODEL = 2, 512, 768
HQ_LOCAL, DH = 8, 64
DQ = HQ_LOCAL * DH
ROWS = B * SQ
CHUNK = ROWS // N_DEV
NC = 3
CW = D_MODEL // NC


def kernel(x, Wq, K_ext, V_ext, Wo):
    def body(x_ref, wq_ref, k_ref, v_ref, wo_ref, out_ref,
             acc_ref, rb1_ref, rb2_ref, rb3_ref, send_sems, recv_sems):
        r = lax.axis_index("i")
        yb = lax.rem(lax.div(r, 2), 2)
        xb = lax.rem(lax.rem(r, 2) + yb, 2)
        zb = lax.div(r, 4)
        px = r + 1 - 2 * lax.rem(r, 2)
        py = 4 * zb + 3 - lax.rem(r, 4)
        pz = lax.rem(r + 4, N_DEV)

        barrier_sem = pltpu.get_barrier_semaphore()
        for nbr in (px, py, pz):
            pl.semaphore_signal(
                barrier_sem, inc=1,
                device_id=(nbr,), device_id_type=pl.DeviceIdType.MESH,
            )
        pl.semaphore_wait(barrier_sem, 3)

        wq = wq_ref[:, pl.ds(r * DQ, DQ)].astype(jnp.bfloat16)
        wo = wo_ref[pl.ds(r * DQ, DQ), :].astype(jnp.bfloat16)
        qi = lax.broadcasted_iota(jnp.int32, (SQ, SQ), 0)
        ki = lax.broadcasted_iota(jnp.int32, (SQ, SQ), 1)
        mask = (jnp.abs(qi - ki) <= 128) | (ki < 32) | (qi < 32)
        bias = jnp.where(mask, 0.0, -1e9).astype(jnp.float32)

        def compute_batch(b):
            xv = x_ref[pl.ds(b, 1), :, :].reshape(SQ, D_MODEL)
            q = lax.dot_general(
                xv.astype(jnp.bfloat16), wq, (((1,), (0,)), ((), ())),
                preferred_element_type=jnp.float32,
            )
            q = (q * 0.125).astype(jnp.bfloat16)
            heads = []
            for h in range(HQ_LOCAL):
                qbh = q[:, h * DH:(h + 1) * DH]
                kbh = k_ref[pl.ds(b, 1), :, h * DH:(h + 1) * DH].reshape(SQ, DH)
                s = lax.dot_general(
                    qbh, kbh.astype(jnp.bfloat16), (((1,), (1,)), ((), ())),
                    preferred_element_type=jnp.float32,
                ) + bias
                w = jnp.exp(s)
                recip = 1.0 / jnp.sum(w, axis=1, keepdims=True)
                vbh = v_ref[pl.ds(b, 1), :, h * DH:(h + 1) * DH].reshape(SQ, DH)
                ctx_h = lax.dot_general(
                    w.astype(jnp.bfloat16), vbh.astype(jnp.bfloat16),
                    (((1,), (0,)), ((), ())),
                    preferred_element_type=jnp.float32,
                ) * recip
                heads.append(ctx_h.astype(jnp.bfloat16))
            ctx = jnp.concatenate(heads, axis=1)
            part = lax.dot_general(
                ctx, wo, (((1,), (0,)), ((), ())),
                preferred_element_type=jnp.float32,
            )
            rows = pl.ds(b * SQ, SQ)
            for g in range(NC):
                acc_ref[g, rows, :] = (
                    part[:, g * CW:(g + 1) * CW].astype(jnp.bfloat16))

        axes = [(px, xb), (py, yb), (pz, zb)]
        rbufs = [rb1_ref, rb2_ref, rb3_ref]

        class Color:
            def __init__(self, g):
                self.g = g
                a = [axes[(g + k) % NC] for k in range(NC)]
                self.p = [a[0][0], a[1][0], a[2][0]]
                b1, b2, b3 = a[0][1], a[1][1], a[2][1]
                self.k = [b1 * (4 * CHUNK), None, None]
                self.gv = [(1 - b1) * (4 * CHUNK), None, None]
                self.k[1] = self.k[0] + b2 * (2 * CHUNK)
                self.gv[1] = self.k[0] + (1 - b2) * (2 * CHUNK)
                self.k[2] = self.k[1] + b3 * CHUNK
                self.gv[2] = self.k[1] + (1 - b3) * CHUNK
                self.n = [4 * CHUNK, 2 * CHUNK, CHUNK]
                self.rs = [None, None, None]
                self.ag = [None, None, None]

            def rs_start(self, s):
                self.rs[s] = pltpu.make_async_remote_copy(
                    src_ref=acc_ref.at[self.g, pl.ds(self.gv[s], self.n[s]), :],
                    dst_ref=rbufs[s].at[self.g, pl.ds(0, self.n[s]), :],
                    send_sem=send_sems.at[self.g * 6 + s],
                    recv_sem=recv_sems.at[self.g * 6 + s],
                    device_id=(self.p[s],),
                    device_id_type=pl.DeviceIdType.MESH,
                )
                self.rs[s].start()

            def rs_finish(self, s):
                self.rs[s].wait()
                rows = pl.ds(self.k[s], self.n[s])
                acc_ref[self.g, rows, :] = (
                    acc_ref[self.g, rows, :]
                    + rbufs[s][self.g, pl.ds(0, self.n[s]), :])

            def ag_start(self, s):
                rows = pl.ds(self.k[2 - s], self.n[2 - s])
                self.ag[s] = pltpu.make_async_remote_copy(
                    src_ref=acc_ref.at[self.g, rows, :],
                    dst_ref=acc_ref.at[self.g, rows, :],
                    send_sem=send_sems.at[self.g * 6 + 3 + s],
                    recv_sem=recv_sems.at[self.g * 6 + 3 + s],
                    device_id=(self.p[2 - s],),
                    device_id_type=pl.DeviceIdType.MESH,
                )
                self.ag[s].start()

            def ag_finish(self, s):
                self.ag[s].wait()

            def store(self):
                out_ref[:, :, pl.ds(self.g * CW, CW)] = (
                    acc_ref[self.g, :, :].astype(jnp.float32)
                    .reshape(B, SQ, CW))

        c0, c1, c2 = Color(0), Color(1), Color(2)

        compute_batch(1 - xb)
        c0.rs_start(0)
        compute_batch(xb)
        c1.rs_start(0)
        c2.rs_start(0)
        c0.rs_finish(0)
        c0.rs_start(1)
        c1.rs_finish(0)
        c1.rs_start(1)
        c2.rs_finish(0)
        c2.rs_start(1)
        c0.rs_finish(1)
        c0.rs_start(2)
        c1.rs_finish(1)
        c1.rs_start(2)
        c2.rs_finish(1)
        c2.rs_start(2)
        c0.rs_finish(2)
        c0.ag_start(0)
        c1.rs_finish(2)
        c1.ag_start(0)
        c2.rs_finish(2)
        c2.ag_start(0)
        c0.ag_finish(0)
        c0.ag_start(1)
        c1.ag_finish(0)
        c1.ag_start(1)
        c2.ag_finish(0)
        c2.ag_start(1)
        c0.ag_finish(1)
        c0.ag_start(2)
        c1.ag_finish(1)
        c1.ag_start(2)
        c2.ag_finish(1)
        c2.ag_start(2)
        c0.ag_finish(2)
        c0.store()
        c1.ag_finish(2)
        c1.store()
        c2.ag_finish(2)
        c2.store()

    kv_shape = (B, SQ, HQ_LOCAL * DH)
    return pl.pallas_call(
        body,
        out_shape=jax.ShapeDtypeStruct((B, SQ, D_MODEL), jnp.float32),
        in_specs=[pl.BlockSpec(memory_space=pltpu.VMEM)] * 5,
        out_specs=pl.BlockSpec(memory_space=pltpu.VMEM),
        scratch_shapes=[
            pltpu.VMEM((NC, ROWS, CW), jnp.bfloat16),
            pltpu.VMEM((NC, 4 * CHUNK, CW), jnp.bfloat16),
            pltpu.VMEM((NC, 2 * CHUNK, CW), jnp.bfloat16),
            pltpu.VMEM((NC, CHUNK, CW), jnp.bfloat16),
            pltpu.SemaphoreType.DMA((18,)),
            pltpu.SemaphoreType.DMA((18,)),
        ],
        compiler_params=pltpu.CompilerParams(
            collective_id=0, vmem_limit_bytes=96 * 1024 * 1024,
        ),
    )(x, Wq, K_ext.reshape(kv_shape), V_ext.reshape(kv_shape), Wo)


# device time: 52448 ns/iter; 2.4758x vs baseline; 1.0076x over previous
import jax
import jax.numpy as jnp
from jax import lax
from jax.experimental import pallas as pl
from jax.experimental.pallas import tpu as pltpu

N_DEV = 8
B, SQ, D_MODEL = 2, 512, 768
HQ_LOCAL, DH = 8, 64
DQ = HQ_LOCAL * DH
ROWS = B * SQ
CHUNK = ROWS // N_DEV
NC = 3
CW = D_MODEL // NC


def kernel(x, Wq, K_ext, V_ext, Wo):
    def body(x_ref, wq_ref, k_ref, v_ref, wo_ref, out_ref,
             acc_ref, rb1_ref, rb2_ref, rb3_ref, send_sems, recv_sems):
        r = lax.axis_index("i")
        yb = lax.rem(lax.div(r, 2), 2)
        xb = lax.rem(lax.rem(r, 2) + yb, 2)
        zb = lax.div(r, 4)
        px = r + 1 - 2 * lax.rem(r, 2)
        py = 4 * zb + 3 - lax.rem(r, 4)
        pz = lax.rem(r + 4, N_DEV)

        barrier_sem = pltpu.get_barrier_semaphore()
        for nbr in (px, py, pz):
            pl.semaphore_signal(
                barrier_sem, inc=1,
                device_id=(nbr,), device_id_type=pl.DeviceIdType.MESH,
            )
        pl.semaphore_wait(barrier_sem, 3)

        wq = wq_ref[:, pl.ds(r * DQ, DQ)].astype(jnp.bfloat16)
        wo = wo_ref[pl.ds(r * DQ, DQ), :].astype(jnp.bfloat16)
        qi = lax.broadcasted_iota(jnp.int32, (SQ, SQ), 0)
        ki = lax.broadcasted_iota(jnp.int32, (SQ, SQ), 1)
        mask = (jnp.abs(qi - ki) <= 128) | (ki < 32) | (qi < 32)
        bias = jnp.where(mask, 0.0, -1e9).astype(jnp.float32)

        def compute_batch(b):
            xv = x_ref[pl.ds(b, 1), :, :].reshape(SQ, D_MODEL)
            q = lax.dot_general(
                xv.astype(jnp.bfloat16), wq, (((1,), (0,)), ((), ())),
                preferred_element_type=jnp.float32,
            )
            q = (q * 0.125).astype(jnp.bfloat16)
            heads = []
            for h in range(HQ_LOCAL):
                qbh = q[:, h * DH:(h + 1) * DH]
                kbh = k_ref[pl.ds(b, 1), :, h * DH:(h + 1) * DH].reshape(SQ, DH)
                s = lax.dot_general(
                    qbh, kbh, (((1,), (1,)), ((), ())),
                    preferred_element_type=jnp.float32,
                ) + bias
                w = jnp.exp(s)
                recip = 1.0 / jnp.sum(w, axis=1, keepdims=True)
                vbh = v_ref[pl.ds(b, 1), :, h * DH:(h + 1) * DH].reshape(SQ, DH)
                ctx_h = lax.dot_general(
                    w.astype(jnp.bfloat16), vbh,
                    (((1,), (0,)), ((), ())),
                    preferred_element_type=jnp.float32,
                ) * recip
                heads.append(ctx_h.astype(jnp.bfloat16))
            ctx = jnp.concatenate(heads, axis=1)
            part = lax.dot_general(
                ctx, wo, (((1,), (0,)), ((), ())),
                preferred_element_type=jnp.float32,
            )
            rows = pl.ds(b * SQ, SQ)
            for g in range(NC):
                acc_ref[g, rows, :] = (
                    part[:, g * CW:(g + 1) * CW].astype(jnp.bfloat16))

        axes = [(px, xb), (py, yb), (pz, zb)]
        rbufs = [rb1_ref, rb2_ref, rb3_ref]

        class Color:
            def __init__(self, g):
                self.g = g
                a = [axes[(g + k) % NC] for k in range(NC)]
                self.p = [a[0][0], a[1][0], a[2][0]]
                b1, b2, b3 = a[0][1], a[1][1], a[2][1]
                self.k = [b1 * (4 * CHUNK), None, None]
                self.gv = [(1 - b1) * (4 * CHUNK), None, None]
                self.k[1] = self.k[0] + b2 * (2 * CHUNK)
                self.gv[1] = self.k[0] + (1 - b2) * (2 * CHUNK)
                self.k[2] = self.k[1] + b3 * CHUNK
                self.gv[2] = self.k[1] + (1 - b3) * CHUNK
                self.n = [4 * CHUNK, 2 * CHUNK, CHUNK]
                self.rs = [None, None, None]
                self.ag = [None, None, None]

            def rs_start(self, s):
                self.rs[s] = pltpu.make_async_remote_copy(
                    src_ref=acc_ref.at[self.g, pl.ds(self.gv[s], self.n[s]), :],
                    dst_ref=rbufs[s].at[self.g, pl.ds(0, self.n[s]), :],
                    send_sem=send_sems.at[self.g * 6 + s],
                    recv_sem=recv_sems.at[self.g * 6 + s],
                    device_id=(self.p[s],),
                    device_id_type=pl.DeviceIdType.MESH,
                )
                self.rs[s].start()

            def rs_finish(self, s):
                self.rs[s].wait()
                rows = pl.ds(self.k[s], self.n[s])
                acc_ref[self.g, rows, :] = (
                    acc_ref[self.g, rows, :]
                    + rbufs[s][self.g, pl.ds(0, self.n[s]), :])

            def ag_start(self, s):
                rows = pl.ds(self.k[2 - s], self.n[2 - s])
                self.ag[s] = pltpu.make_async_remote_copy(
                    src_ref=acc_ref.at[self.g, rows, :],
                    dst_ref=acc_ref.at[self.g, rows, :],
                    send_sem=send_sems.at[self.g * 6 + 3 + s],
                    recv_sem=recv_sems.at[self.g * 6 + 3 + s],
                    device_id=(self.p[2 - s],),
                    device_id_type=pl.DeviceIdType.MESH,
                )
                self.ag[s].start()

            def ag_finish(self, s):
                self.ag[s].wait()

            def store(self):
                out_ref[:, :, pl.ds(self.g * CW, CW)] = (
                    acc_ref[self.g, :, :].astype(jnp.float32)
                    .reshape(B, SQ, CW))

        c0, c1, c2 = Color(0), Color(1), Color(2)

        compute_batch(1 - xb)
        c0.rs_start(0)
        compute_batch(xb)
        c1.rs_start(0)
        c2.rs_start(0)
        c0.rs_finish(0)
        c0.rs_start(1)
        c1.rs_finish(0)
        c1.rs_start(1)
        c2.rs_finish(0)
        c2.rs_start(1)
        c0.rs_finish(1)
        c0.rs_start(2)
        c1.rs_finish(1)
        c1.rs_start(2)
        c2.rs_finish(1)
        c2.rs_start(2)
        c0.rs_finish(2)
        c0.ag_start(0)
        c1.rs_finish(2)
        c1.ag_start(0)
        c2.rs_finish(2)
        c2.ag_start(0)
        c0.ag_finish(0)
        c0.ag_start(1)
        c1.ag_finish(0)
        c1.ag_start(1)
        c2.ag_finish(0)
        c2.ag_start(1)
        c0.ag_finish(1)
        c0.ag_start(2)
        c1.ag_finish(1)
        c1.ag_start(2)
        c2.ag_finish(1)
        c2.ag_start(2)
        c0.ag_finish(2)
        c0.store()
        c1.ag_finish(2)
        c1.store()
        c2.ag_finish(2)
        c2.store()

    kv_shape = (B, SQ, HQ_LOCAL * DH)
    return pl.pallas_call(
        body,
        out_shape=jax.ShapeDtypeStruct((B, SQ, D_MODEL), jnp.float32),
        in_specs=[pl.BlockSpec(memory_space=pltpu.VMEM)] * 5,
        out_specs=pl.BlockSpec(memory_space=pltpu.VMEM),
        scratch_shapes=[
            pltpu.VMEM((NC, ROWS, CW), jnp.bfloat16),
            pltpu.VMEM((NC, 4 * CHUNK, CW), jnp.bfloat16),
            pltpu.VMEM((NC, 2 * CHUNK, CW), jnp.bfloat16),
            pltpu.VMEM((NC, CHUNK, CW), jnp.bfloat16),
            pltpu.SemaphoreType.DMA((18,)),
            pltpu.SemaphoreType.DMA((18,)),
        ],
        compiler_params=pltpu.CompilerParams(
            collective_id=0, vmem_limit_bytes=96 * 1024 * 1024,
        ),
    )(x, Wq,
      K_ext.reshape(kv_shape).astype(jnp.bfloat16),
      V_ext.reshape(kv_shape).astype(jnp.bfloat16), Wo)


# device time: 49972 ns/iter; 2.5985x vs baseline; 1.0495x over previous
import jax
import jax.numpy as jnp
from jax import lax
from jax.experimental import pallas as pl
from jax.experimental.pallas import tpu as pltpu

N_DEV = 8
B, SQ, D_MODEL = 2, 512, 768
HQ_LOCAL, DH = 8, 64
DQ = HQ_LOCAL * DH
ROWS = B * SQ
CHUNK = ROWS // N_DEV
NC = 3
CW = D_MODEL // NC


def kernel(x, Wq, K_ext, V_ext, Wo):
    def body(x_ref, wq_ref, k_ref, v_ref, wo_ref, out_ref,
             acc_ref, rb1_ref, rb2_ref, arb_ref, send_sems, recv_sems):
        r = lax.axis_index("i")
        yb = lax.rem(lax.div(r, 2), 2)
        xb = lax.rem(lax.rem(r, 2) + yb, 2)
        zb = lax.div(r, 4)
        px = r + 1 - 2 * lax.rem(r, 2)
        py = 4 * zb + 3 - lax.rem(r, 4)
        pz = lax.rem(r + 4, N_DEV)

        barrier_sem = pltpu.get_barrier_semaphore()
        for nbr in (px, py, pz):
            pl.semaphore_signal(
                barrier_sem, inc=1,
                device_id=(nbr,), device_id_type=pl.DeviceIdType.MESH,
            )
        pl.semaphore_wait(barrier_sem, 3)

        wq = wq_ref[:, pl.ds(r * DQ, DQ)].astype(jnp.bfloat16)
        wo = wo_ref[pl.ds(r * DQ, DQ), :].astype(jnp.bfloat16)
        qi = lax.broadcasted_iota(jnp.int32, (SQ, SQ), 0)
        ki = lax.broadcasted_iota(jnp.int32, (SQ, SQ), 1)
        mask = (jnp.abs(qi - ki) <= 128) | (ki < 32) | (qi < 32)
        bias = jnp.where(mask, 0.0, -1e9).astype(jnp.float32)

        def compute_batch(b):
            xv = x_ref[pl.ds(b, 1), :, :].reshape(SQ, D_MODEL)
            q = lax.dot_general(
                xv.astype(jnp.bfloat16), wq, (((1,), (0,)), ((), ())),
                preferred_element_type=jnp.float32,
            )
            q = (q * 0.125).astype(jnp.bfloat16)
            heads = []
            for h in range(HQ_LOCAL):
                qbh = q[:, h * DH:(h + 1) * DH]
                kbh = k_ref[pl.ds(b, 1), :, h * DH:(h + 1) * DH].reshape(SQ, DH)
                s = lax.dot_general(
                    qbh, kbh, (((1,), (1,)), ((), ())),
                    preferred_element_type=jnp.float32,
                ) + bias
                w = jnp.exp(s)
                recip = 1.0 / jnp.sum(w, axis=1, keepdims=True)
                vbh = v_ref[pl.ds(b, 1), :, h * DH:(h + 1) * DH].reshape(SQ, DH)
                ctx_h = lax.dot_general(
                    w.astype(jnp.bfloat16), vbh,
                    (((1,), (0,)), ((), ())),
                    preferred_element_type=jnp.float32,
                ) * recip
                heads.append(ctx_h.astype(jnp.bfloat16))
            ctx = jnp.concatenate(heads, axis=1)
            part = lax.dot_general(
                ctx, wo, (((1,), (0,)), ((), ())),
                preferred_element_type=jnp.float32,
            )
            rows = pl.ds(b * SQ, SQ)
            for g in range(NC):
                acc_ref[g, rows, :] = (
                    part[:, g * CW:(g + 1) * CW].astype(jnp.bfloat16))

        axes = [(px, xb), (py, yb), (pz, zb)]
        rbufs = [rb1_ref, rb2_ref]

        class Color:
            def __init__(self, g):
                self.g = g
                a = [axes[(g + k) % NC] for k in range(NC)]
                self.p = [a[0][0], a[1][0], a[2][0]]
                b1, b2, b3 = a[0][1], a[1][1], a[2][1]
                self.k = [b1 * (4 * CHUNK), None, None]
                self.gv = [(1 - b1) * (4 * CHUNK), None, None]
                self.k[1] = self.k[0] + b2 * (2 * CHUNK)
                self.gv[1] = self.k[0] + (1 - b2) * (2 * CHUNK)
                self.k[2] = self.k[1] + b3 * CHUNK
                self.gv[2] = self.k[1] + (1 - b3) * CHUNK
                self.n = [4 * CHUNK, 2 * CHUNK, CHUNK]
                self.rs = [None, None, None]
                self.ag = [None, None, None]
                self.ar = None

            def rs_start(self, s):
                self.rs[s] = pltpu.make_async_remote_copy(
                    src_ref=acc_ref.at[self.g, pl.ds(self.gv[s], self.n[s]), :],
                    dst_ref=rbufs[s].at[self.g, pl.ds(0, self.n[s]), :],
                    send_sem=send_sems.at[self.g * 6 + s],
                    recv_sem=recv_sems.at[self.g * 6 + s],
                    device_id=(self.p[s],),
                    device_id_type=pl.DeviceIdType.MESH,
                )
                self.rs[s].start()

            def rs_finish(self, s):
                self.rs[s].wait()
                rows = pl.ds(self.k[s], self.n[s])
                acc_ref[self.g, rows, :] = (
                    acc_ref[self.g, rows, :]
                    + rbufs[s][self.g, pl.ds(0, self.n[s]), :])

            def ar_start(self):
                rows = pl.ds(self.k[1], 2 * CHUNK)
                self.ar = pltpu.make_async_remote_copy(
                    src_ref=acc_ref.at[self.g, rows, :],
                    dst_ref=arb_ref.at[self.g, :, :],
                    send_sem=send_sems.at[self.g * 6 + 2],
                    recv_sem=recv_sems.at[self.g * 6 + 2],
                    device_id=(self.p[2],),
                    device_id_type=pl.DeviceIdType.MESH,
                )
                self.ar.start()

            def ar_finish(self):
                self.ar.wait()
                rows = pl.ds(self.k[1], 2 * CHUNK)
                acc_ref[self.g, rows, :] = (
                    acc_ref[self.g, rows, :] + arb_ref[self.g, :, :])

            def ag_start(self, s):
                rows = pl.ds(self.k[2 - s], self.n[2 - s])
                self.ag[s] = pltpu.make_async_remote_copy(
                    src_ref=acc_ref.at[self.g, rows, :],
                    dst_ref=acc_ref.at[self.g, rows, :],
                    send_sem=send_sems.at[self.g * 6 + 3 + s],
                    recv_sem=recv_sems.at[self.g * 6 + 3 + s],
                    device_id=(self.p[2 - s],),
                    device_id_type=pl.DeviceIdType.MESH,
                )
                self.ag[s].start()

            def ag_finish(self, s):
                self.ag[s].wait()

            def store(self):
                out_ref[:, :, pl.ds(self.g * CW, CW)] = (
                    acc_ref[self.g, :, :].astype(jnp.float32)
                    .reshape(B, SQ, CW))

        c0, c1, c2 = Color(0), Color(1), Color(2)

        compute_batch(1 - xb)
        c0.rs_start(0)
        compute_batch(xb)
        c1.rs_start(0)
        c2.rs_start(0)
        c0.rs_finish(0)
        c0.rs_start(1)
        c1.rs_finish(0)
        c1.rs_start(1)
        c2.rs_finish(0)
        c2.rs_start(1)
        c0.rs_finish(1)
        c0.ar_start()
        c1.rs_finish(1)
        c1.ar_start()
        c2.rs_finish(1)
        c2.ar_start()
        c0.ar_finish()
        c0.ag_start(1)
        c1.ar_finish()
        c1.ag_start(1)
        c2.ar_finish()
        c2.ag_start(1)
        c0.ag_finish(1)
        c0.ag_start(2)
        c1.ag_finish(1)
        c1.ag_start(2)
        c2.ag_finish(1)
        c2.ag_start(2)
        c0.ag_finish(2)
        c0.store()
        c1.ag_finish(2)
        c1.store()
        c2.ag_finish(2)
        c2.store()

    kv_shape = (B, SQ, HQ_LOCAL * DH)
    return pl.pallas_call(
        body,
        out_shape=jax.ShapeDtypeStruct((B, SQ, D_MODEL), jnp.float32),
        in_specs=[pl.BlockSpec(memory_space=pltpu.VMEM)] * 5,
        out_specs=pl.BlockSpec(memory_space=pltpu.VMEM),
        scratch_shapes=[
            pltpu.VMEM((NC, ROWS, CW), jnp.bfloat16),
            pltpu.VMEM((NC, 4 * CHUNK, CW), jnp.bfloat16),
            pltpu.VMEM((NC, 2 * CHUNK, CW), jnp.bfloat16),
            pltpu.VMEM((NC, 2 * CHUNK, CW), jnp.bfloat16),
            pltpu.SemaphoreType.DMA((18,)),
            pltpu.SemaphoreType.DMA((18,)),
        ],
        compiler_params=pltpu.CompilerParams(
            collective_id=0, vmem_limit_bytes=96 * 1024 * 1024,
        ),
    )(x, Wq,
      K_ext.reshape(kv_shape).astype(jnp.bfloat16),
      V_ext.reshape(kv_shape).astype(jnp.bfloat16), Wo)
